# baseline (speedup 1.0000x reference)
import functools

import jax
import jax.numpy as jnp
import numpy as np
from jax.experimental import pallas as pl
from jax.experimental.pallas import tpu as pltpu

B, A, OBS_DIM = 8, 32, 256
THOUGHT, ACT_DIM, ATT_EMB = 128, 32, 64
M_GROUP, THRESH = 8, 0.4
H = THOUGHT // 2
N = B * A

_HI = jax.lax.Precision.HIGHEST
_BIG = 1e30


def _mm(a, b, prec=None):
    return jax.lax.dot_general(a, b, (((1,), (0,)), ((), ())), precision=prec,
                               preferred_element_type=jnp.float32)


def _mmT(a, b, prec=None):
    return jax.lax.dot_general(a, b, (((1,), (1,)), ((), ())), precision=prec,
                               preferred_element_type=jnp.float32)


def _mTm(a, b, prec=None):
    return jax.lax.dot_general(a, b, (((0,), (0,)), ((), ())), precision=prec,
                               preferred_element_type=jnp.float32)


def _ln_in(x, g, b):
    m = jnp.mean(x, axis=-1, keepdims=True)
    v = jnp.mean((x - m) ** 2, axis=-1, keepdims=True)
    return (x - m) * jax.lax.rsqrt(v + 1e-5) * g + b


def _fused_kernel(obs_ref, w1t_ref, b1_ref, g1_ref, bt1_ref,
                  w2t_ref, b2_ref, g2_ref, bt2_ref,
                  aw1t_ref, ab1_ref, aw2t_ref, ab2_ref, aw3t_ref, ab3_ref,
                  wih_ref, whh_ref, bc_ref,
                  c1t_ref, c1b_ref, c1g_ref, c1bt_ref,
                  c2t_ref, c2b_ref, c2g_ref, c2bt_ref,
                  pm_ref,
                  act_out, grp_out, ip_out, isin_out, nt_out, ct_out,
                  nt_ref, slot_ref, ii_ref):
    x = _mm(obs_ref[...], w1t_ref[...]) + b1_ref[...]
    x = jax.nn.relu(_ln_in(x, g1_ref[...], bt1_ref[...]))
    ct = _ln_in(_mm(x, w2t_ref[...]) + b2_ref[...], g2_ref[...], bt2_ref[...])
    ct_out[...] = ct

    h = jax.nn.relu(_mm(ct, aw1t_ref[...]) + ab1_ref[...])
    h = jax.nn.relu(_mm(h, aw2t_ref[...]) + ab2_ref[...])
    ip = jax.nn.sigmoid(_mm(h, aw3t_ref[...]) + ab3_ref[...])
    ip_out[...] = ip
    isin = jnp.where(ip > THRESH, 1.0, 0.0).astype(jnp.float32)
    isin_out[...] = isin

    pm = pm_ref[...]
    ct_im = _mm(pm, ct, _HI)
    sq_b = jnp.sum(ct * ct, axis=-1, keepdims=True)
    sq_im = jnp.sum(ct_im * ct_im, axis=-1, keepdims=True)
    ones = jnp.ones((N, 1), jnp.float32)
    xa = jnp.concatenate([ct_im * (-2.0), sq_im, ones], axis=1)
    ya = jnp.concatenate([ct, ones, sq_b], axis=1)
    dists = _mmT(xa, ya, _HI)
    r256 = jax.lax.broadcasted_iota(jnp.int32, (N, N), 0)
    c256 = jax.lax.broadcasted_iota(jnp.int32, (N, N), 1)
    offblock = jnp.bitwise_and(r256, 7) != jax.lax.shift_right_logical(c256, 5)
    dists = jnp.where(offblock, _BIG, dists)

    c32 = jax.lax.broadcasted_iota(jnp.int32, (N, A), 1)
    grp_im = jnp.zeros((N, A), jnp.float32)
    for _ in range(M_GROUP):
        idx = jnp.argmin(dists, axis=-1).astype(jnp.int32)[:, None]
        dists = jnp.where(c256 == idx, _BIG, dists)
        j = jnp.bitwise_and(idx, 31)
        grp_im = grp_im + jnp.where(c32 == j, 1.0, 0.0)

    tri = jnp.where(
        jax.lax.broadcasted_iota(jnp.int32, (A, A), 0)
        <= jax.lax.broadcasted_iota(jnp.int32, (A, A), 1), 1.0, 0.0)
    pos = _mm(grp_im, tri)
    slot_ref[...] = pos * grp_im - (1.0 - grp_im)
    ii_im = _mm(pm, isin)
    ii_ref[...] = ii_im

    grp_out[...] = _mTm(pm, grp_im * ii_im)

    nt_ref[...] = ct

    r64 = jax.lax.broadcasted_iota(jnp.int32, (8 * M_GROUP, N), 0)
    c64 = jax.lax.broadcasted_iota(jnp.int32, (8 * M_GROUP, N), 1)
    tau_f = jax.lax.shift_right_logical(r64, 3).astype(jnp.float32)
    blockdiag = jnp.bitwise_and(r64, 7) == jax.lax.shift_right_logical(c64, 5)
    lane512 = jax.lax.broadcasted_iota(jnp.int32, (1, 4 * THOUGHT), 1)
    fwd512 = jnp.bitwise_and(jax.lax.shift_right_logical(lane512, 6), 1) == 0
    lane128 = jax.lax.broadcasted_iota(jnp.int32, (1, THOUGHT), 1)
    fwd128 = lane128 < H

    wih = wih_ref[...]
    whh = whh_ref[...]
    bc = bc_ref[...]

    def step(i, carry):
        sl = slot_ref[pl.ds(8 * i, 8), :]
        ii = ii_ref[pl.ds(8 * i, 8), :]
        nt = nt_ref[...]

        sl_exp = jnp.tile(jnp.tile(sl, (8, 1)), (1, 8))
        mt = jnp.where((sl_exp - 1.0 == tau_f) & blockdiag, 1.0, 0.0)

        th = _mm(mt, nt, _HI)
        xall = _mm(th, wih) + bc

        c = jnp.zeros((8, THOUGHT), jnp.float32)
        hh = jnp.zeros((8, THOUGHT), jnp.float32)
        hs = []
        for t in range(M_GROUP):
            xf = xall[8 * t:8 * t + 8, :]
            xr = xall[8 * (7 - t):8 * (7 - t) + 8, :]
            g = _mm(hh, whh) + jnp.where(fwd512, xf, xr)
            ig = jax.nn.sigmoid(g[:, 0:128])
            fg = jax.nn.sigmoid(g[:, 128:256])
            gg = jnp.tanh(g[:, 256:384])
            og = jax.nn.sigmoid(g[:, 384:512])
            c = fg * c + ig * gg
            hh = og * jnp.tanh(c)
            hs.append(hh)
        integ = jnp.concatenate(
            [jnp.where(fwd128, hs[t], hs[7 - t]) for t in range(M_GROUP)],
            axis=0)

        delta = (integ - th) * jnp.tile(ii, (8, 1))
        nt_ref[...] = nt + _mTm(mt, delta, _HI)
        return carry

    jax.lax.fori_loop(0, A, step, 0)

    nt = nt_ref[...]
    nt_out[...] = nt

    xc = jax.nn.relu(jnp.concatenate([ct, nt], axis=1))
    y = _ln_in(_mm(xc, c1t_ref[...]) + c1b_ref[...], c1g_ref[...], c1bt_ref[...])
    z = _ln_in(_mm(y, c2t_ref[...]) + c2b_ref[...], c2g_ref[...], c2bt_ref[...])
    act_out[...] = jnp.tanh(z)


@jax.jit
def kernel(obs, a1_w1, a1_b1, a1_g1, a1_bt1, a1_w2, a1_b2, a1_g2, a1_bt2,
           a2_w1, a2_b1, a2_g1, a2_bt1, a2_w2, a2_b2, a2_g2, a2_bt2,
           att_w1, att_b1, att_w2, att_b2, att_w3, att_b3,
           wih_f, whh_f, bih_f, bhh_f, wih_r, whh_r, bih_r, bhh_r):
    obs_f = obs.reshape(N, OBS_DIM)

    def row(v):
        return v.reshape(1, -1)

    wft, wrt = wih_f.T, wih_r.T
    wih_c = jnp.concatenate(
        [jnp.concatenate([wft[:, 64 * k:64 * (k + 1)],
                          wrt[:, 64 * k:64 * (k + 1)]], axis=1)
         for k in range(4)], axis=1)
    hft, hrt = whh_f.T, whh_r.T
    zz = jnp.zeros((H, H), jnp.float32)
    whh_c = jnp.concatenate(
        [jnp.concatenate(
            [jnp.concatenate([hft[:, 64 * k:64 * (k + 1)], zz], axis=1),
             jnp.concatenate([zz, hrt[:, 64 * k:64 * (k + 1)]], axis=1)],
            axis=0)
         for k in range(4)], axis=1)
    bf = bih_f + bhh_f
    br = bih_r + bhh_r
    bc = jnp.concatenate(
        [jnp.concatenate([bf[64 * k:64 * (k + 1)], br[64 * k:64 * (k + 1)]])
         for k in range(4)]).reshape(1, -1)

    pm_np = np.zeros((N, N), np.float32)
    rp = np.arange(N)
    pm_np[rp, 32 * (rp % 8) + rp // 8] = 1.0
    pm = jnp.asarray(pm_np)

    out_shapes = [
        jax.ShapeDtypeStruct((N, ACT_DIM), jnp.float32),
        jax.ShapeDtypeStruct((N, A), jnp.float32),
        jax.ShapeDtypeStruct((N, 1), jnp.float32),
        jax.ShapeDtypeStruct((N, 1), jnp.float32),
        jax.ShapeDtypeStruct((N, THOUGHT), jnp.float32),
        jax.ShapeDtypeStruct((N, THOUGHT), jnp.float32),
    ]
    act, grp, ip, isin, nt, ct = pl.pallas_call(
        _fused_kernel,
        out_shape=out_shapes,
        scratch_shapes=[
            pltpu.VMEM((N, THOUGHT), jnp.float32),
            pltpu.VMEM((N, A), jnp.float32),
            pltpu.VMEM((N, 1), jnp.float32),
        ],
    )(obs_f, a1_w1.T, row(a1_b1), row(a1_g1), row(a1_bt1),
      a1_w2.T, row(a1_b2), row(a1_g2), row(a1_bt2),
      att_w1.T, row(att_b1), att_w2.T, row(att_b2), att_w3.T, row(att_b3),
      wih_c, whh_c, bc,
      a2_w1.T, row(a2_b1), row(a2_g1), row(a2_bt1),
      a2_w2.T, row(a2_b2), row(a2_g2), row(a2_bt2),
      pm)

    return (act.reshape(B, A, ACT_DIM), grp.reshape(B, A, A),
            ip.reshape(B, A), (isin > 0.5).reshape(B, A),
            nt.reshape(B, A, THOUGHT), ct.reshape(B, A, THOUGHT))


# speedup vs baseline: 1.0585x; 1.0585x over previous
import functools

import jax
import jax.numpy as jnp
import numpy as np
from jax.experimental import pallas as pl
from jax.experimental.pallas import tpu as pltpu

B, A, OBS_DIM = 8, 32, 256
THOUGHT, ACT_DIM, ATT_EMB = 128, 32, 64
M_GROUP, THRESH = 8, 0.4
H = THOUGHT // 2
N = B * A

_HI = jax.lax.Precision.HIGHEST
_BIG = 1e30


def _mm(a, b, prec=None):
    return jax.lax.dot_general(a, b, (((1,), (0,)), ((), ())), precision=prec,
                               preferred_element_type=jnp.float32)


def _mmT(a, b, prec=None):
    return jax.lax.dot_general(a, b, (((1,), (1,)), ((), ())), precision=prec,
                               preferred_element_type=jnp.float32)


def _mTm(a, b, prec=None):
    return jax.lax.dot_general(a, b, (((0,), (0,)), ((), ())), precision=prec,
                               preferred_element_type=jnp.float32)


def _ln_in(x, g, b):
    m = jnp.mean(x, axis=-1, keepdims=True)
    v = jnp.mean((x - m) ** 2, axis=-1, keepdims=True)
    return (x - m) * jax.lax.rsqrt(v + 1e-5) * g + b


def _fused_kernel(obs_ref, w1t_ref, b1_ref, g1_ref, bt1_ref,
                  w2t_ref, b2_ref, g2_ref, bt2_ref,
                  aw1t_ref, ab1_ref, aw2t_ref, ab2_ref, aw3t_ref, ab3_ref,
                  wih_ref, whh_ref, bc_ref,
                  c1t_ref, c1b_ref, c1g_ref, c1bt_ref,
                  c2t_ref, c2b_ref, c2g_ref, c2bt_ref,
                  pm_ref,
                  act_out, grp_out, ip_out, isin_out, nt_out, ct_out,
                  nt_ref, slot_ref, ii_ref):
    x = _mm(obs_ref[...], w1t_ref[...]) + b1_ref[...]
    x = jax.nn.relu(_ln_in(x, g1_ref[...], bt1_ref[...]))
    ct = _ln_in(_mm(x, w2t_ref[...]) + b2_ref[...], g2_ref[...], bt2_ref[...])
    ct_out[...] = ct

    h = jax.nn.relu(_mm(ct, aw1t_ref[...]) + ab1_ref[...])
    h = jax.nn.relu(_mm(h, aw2t_ref[...]) + ab2_ref[...])
    ip = jax.nn.sigmoid(_mm(h, aw3t_ref[...]) + ab3_ref[...])
    ip_out[...] = ip
    isin = jnp.where(ip > THRESH, 1.0, 0.0).astype(jnp.float32)
    isin_out[...] = isin

    pm = pm_ref[...]
    ct_im = _mm(pm, ct, _HI)
    sq_b = jnp.sum(ct * ct, axis=-1, keepdims=True)
    sq_im = jnp.sum(ct_im * ct_im, axis=-1, keepdims=True)
    ones = jnp.ones((N, 1), jnp.float32)
    xa = jnp.concatenate([ct_im * (-2.0), sq_im, ones], axis=1)
    ya = jnp.concatenate([ct, ones, sq_b], axis=1)
    dists = _mmT(xa, ya, _HI)
    r256 = jax.lax.broadcasted_iota(jnp.int32, (N, N), 0)
    c256 = jax.lax.broadcasted_iota(jnp.int32, (N, N), 1)
    offblock = jnp.bitwise_and(r256, 7) != jax.lax.shift_right_logical(c256, 5)
    dists = jnp.where(offblock, _BIG, dists)

    c32 = jax.lax.broadcasted_iota(jnp.int32, (N, A), 1)
    grp_im = jnp.zeros((N, A), jnp.float32)
    for _ in range(M_GROUP):
        idx = jnp.argmin(dists, axis=-1).astype(jnp.int32)[:, None]
        dists = jnp.where(c256 == idx, _BIG, dists)
        j = jnp.bitwise_and(idx, 31)
        grp_im = grp_im + jnp.where(c32 == j, 1.0, 0.0)

    tri = jnp.where(
        jax.lax.broadcasted_iota(jnp.int32, (A, A), 0)
        <= jax.lax.broadcasted_iota(jnp.int32, (A, A), 1), 1.0, 0.0)
    pos = _mm(grp_im, tri)
    slot_ref[0:N, :] = pos * grp_im - (1.0 - grp_im)
    slot_ref[N:N + 8, :] = jnp.full((8, A), -1.0, jnp.float32)
    ii_im = _mm(pm, isin)
    ii_ref[...] = ii_im

    grp_out[...] = _mTm(pm, grp_im * ii_im)

    nt_ref[...] = ct

    r64 = jax.lax.broadcasted_iota(jnp.int32, (8 * M_GROUP, N), 0)
    c64 = jax.lax.broadcasted_iota(jnp.int32, (8 * M_GROUP, N), 1)
    tau_f = jax.lax.shift_right_logical(r64, 3).astype(jnp.float32)
    blockdiag = jnp.bitwise_and(r64, 7) == jax.lax.shift_right_logical(c64, 5)
    lane512 = jax.lax.broadcasted_iota(jnp.int32, (1, 4 * THOUGHT), 1)
    fwd512 = jnp.bitwise_and(jax.lax.shift_right_logical(lane512, 6), 1) == 0
    lane128 = jax.lax.broadcasted_iota(jnp.int32, (1, THOUGHT), 1)
    fwd128 = lane128 < H

    wih = wih_ref[...]
    whh = whh_ref[...]
    bc = bc_ref[...]

    def build_mt(sl):
        sl_exp = jnp.tile(jnp.tile(sl, (8, 1)), (1, 8))
        return jnp.where((sl_exp - 1.0 == tau_f) & blockdiag, 1.0, 0.0)

    th0 = _mm(build_mt(slot_ref[0:8, :]), ct, _HI)

    def step(i, th):
        sl = slot_ref[pl.ds(8 * i, 8), :]
        sln = slot_ref[pl.ds(8 * i + 8, 8), :]
        ii = ii_ref[pl.ds(8 * i, 8), :]
        nt = nt_ref[...]
        mt = build_mt(sl)
        mtn = build_mt(sln)

        xall = _mm(th, wih) + bc

        c = jnp.zeros((8, THOUGHT), jnp.float32)
        hh = jnp.zeros((8, THOUGHT), jnp.float32)
        hs = []
        for t in range(M_GROUP):
            xf = xall[8 * t:8 * t + 8, :]
            xr = xall[8 * (7 - t):8 * (7 - t) + 8, :]
            g = jnp.where(fwd512, xf, xr)
            if t > 0:
                g = g + _mm(hh, whh)
            ig = jax.nn.sigmoid(g[:, 0:128])
            fg = jax.nn.sigmoid(g[:, 128:256])
            gg = jnp.tanh(g[:, 256:384])
            og = jax.nn.sigmoid(g[:, 384:512])
            c = fg * c + ig * gg if t > 0 else ig * gg
            hh = og * jnp.tanh(c)
            hs.append(hh)
        integ = jnp.concatenate(
            [jnp.where(fwd128, hs[t], hs[7 - t]) for t in range(M_GROUP)],
            axis=0)

        delta = (integ - th) * jnp.tile(ii, (8, 1))
        nt_ref[...] = nt + _mTm(mt, delta, _HI)

        pre = _mm(mtn, nt, _HI)
        cc = _mmT(mtn, mt)
        return pre + _mm(cc, delta, _HI)

    jax.lax.fori_loop(0, A, step, th0)

    nt = nt_ref[...]
    nt_out[...] = nt

    xc = jax.nn.relu(jnp.concatenate([ct, nt], axis=1))
    y = _ln_in(_mm(xc, c1t_ref[...]) + c1b_ref[...], c1g_ref[...], c1bt_ref[...])
    z = _ln_in(_mm(y, c2t_ref[...]) + c2b_ref[...], c2g_ref[...], c2bt_ref[...])
    act_out[...] = jnp.tanh(z)


@jax.jit
def kernel(obs, a1_w1, a1_b1, a1_g1, a1_bt1, a1_w2, a1_b2, a1_g2, a1_bt2,
           a2_w1, a2_b1, a2_g1, a2_bt1, a2_w2, a2_b2, a2_g2, a2_bt2,
           att_w1, att_b1, att_w2, att_b2, att_w3, att_b3,
           wih_f, whh_f, bih_f, bhh_f, wih_r, whh_r, bih_r, bhh_r):
    obs_f = obs.reshape(N, OBS_DIM)

    def row(v):
        return v.reshape(1, -1)

    wft, wrt = wih_f.T, wih_r.T
    wih_c = jnp.concatenate(
        [jnp.concatenate([wft[:, 64 * k:64 * (k + 1)],
                          wrt[:, 64 * k:64 * (k + 1)]], axis=1)
         for k in range(4)], axis=1)
    hft, hrt = whh_f.T, whh_r.T
    zz = jnp.zeros((H, H), jnp.float32)
    whh_c = jnp.concatenate(
        [jnp.concatenate(
            [jnp.concatenate([hft[:, 64 * k:64 * (k + 1)], zz], axis=1),
             jnp.concatenate([zz, hrt[:, 64 * k:64 * (k + 1)]], axis=1)],
            axis=0)
         for k in range(4)], axis=1)
    bf = bih_f + bhh_f
    br = bih_r + bhh_r
    bc = jnp.concatenate(
        [jnp.concatenate([bf[64 * k:64 * (k + 1)], br[64 * k:64 * (k + 1)]])
         for k in range(4)]).reshape(1, -1)

    pm_np = np.zeros((N, N), np.float32)
    rp = np.arange(N)
    pm_np[rp, 32 * (rp % 8) + rp // 8] = 1.0
    pm = jnp.asarray(pm_np)

    out_shapes = [
        jax.ShapeDtypeStruct((N, ACT_DIM), jnp.float32),
        jax.ShapeDtypeStruct((N, A), jnp.float32),
        jax.ShapeDtypeStruct((N, 1), jnp.float32),
        jax.ShapeDtypeStruct((N, 1), jnp.float32),
        jax.ShapeDtypeStruct((N, THOUGHT), jnp.float32),
        jax.ShapeDtypeStruct((N, THOUGHT), jnp.float32),
    ]
    act, grp, ip, isin, nt, ct = pl.pallas_call(
        _fused_kernel,
        out_shape=out_shapes,
        scratch_shapes=[
            pltpu.VMEM((N, THOUGHT), jnp.float32),
            pltpu.VMEM((N + 8, A), jnp.float32),
            pltpu.VMEM((N, 1), jnp.float32),
        ],
    )(obs_f, a1_w1.T, row(a1_b1), row(a1_g1), row(a1_bt1),
      a1_w2.T, row(a1_b2), row(a1_g2), row(a1_bt2),
      att_w1.T, row(att_b1), att_w2.T, row(att_b2), att_w3.T, row(att_b3),
      wih_c, whh_c, bc,
      a2_w1.T, row(a2_b1), row(a2_g1), row(a2_bt1),
      a2_w2.T, row(a2_b2), row(a2_g2), row(a2_bt2),
      pm)

    return (act.reshape(B, A, ACT_DIM), grp.reshape(B, A, A),
            ip.reshape(B, A), (isin > 0.5).reshape(B, A),
            nt.reshape(B, A, THOUGHT), ct.reshape(B, A, THOUGHT))


# speedup vs baseline: 1.0932x; 1.0328x over previous
import functools

import jax
import jax.numpy as jnp
import numpy as np
from jax.experimental import pallas as pl
from jax.experimental.pallas import tpu as pltpu

B, A, OBS_DIM = 8, 32, 256
THOUGHT, ACT_DIM, ATT_EMB = 128, 32, 64
M_GROUP, THRESH = 8, 0.4
H = THOUGHT // 2
N = B * A

_HI = jax.lax.Precision.HIGHEST
_BIG = 1e30


def _mm(a, b, prec=None):
    return jax.lax.dot_general(a, b, (((1,), (0,)), ((), ())), precision=prec,
                               preferred_element_type=jnp.float32)


def _mmT(a, b, prec=None):
    return jax.lax.dot_general(a, b, (((1,), (1,)), ((), ())), precision=prec,
                               preferred_element_type=jnp.float32)


def _mTm(a, b, prec=None):
    return jax.lax.dot_general(a, b, (((0,), (0,)), ((), ())), precision=prec,
                               preferred_element_type=jnp.float32)


def _ln_in(x, g, b):
    m = jnp.mean(x, axis=-1, keepdims=True)
    v = jnp.mean((x - m) ** 2, axis=-1, keepdims=True)
    return (x - m) * jax.lax.rsqrt(v + 1e-5) * g + b


def _fused_kernel(obs_ref, w1t_ref, b1_ref, g1_ref, bt1_ref,
                  w2t_ref, b2_ref, g2_ref, bt2_ref,
                  aw1t_ref, ab1_ref, aw2t_ref, ab2_ref, aw3t_ref, ab3_ref,
                  wih_ref, whh_ref, bc_ref,
                  c1t_ref, c1b_ref, c1g_ref, c1bt_ref,
                  c2t_ref, c2b_ref, c2g_ref, c2bt_ref,
                  pm_ref,
                  act_out, grp_out, ip_out, isin_out, nt_out, ct_out,
                  nt_ref, slot_ref, ii_ref):
    x = _mm(obs_ref[...], w1t_ref[...]) + b1_ref[...]
    x = jax.nn.relu(_ln_in(x, g1_ref[...], bt1_ref[...]))
    ct = _ln_in(_mm(x, w2t_ref[...]) + b2_ref[...], g2_ref[...], bt2_ref[...])
    ct_out[...] = ct

    h = jax.nn.relu(_mm(ct, aw1t_ref[...]) + ab1_ref[...])
    h = jax.nn.relu(_mm(h, aw2t_ref[...]) + ab2_ref[...])
    ip = jax.nn.sigmoid(_mm(h, aw3t_ref[...]) + ab3_ref[...])
    ip_out[...] = ip
    isin = jnp.where(ip > THRESH, 1.0, 0.0).astype(jnp.float32)
    isin_out[...] = isin

    pm = pm_ref[...]
    ct_im = _mm(pm, ct, _HI)
    sq_b = jnp.sum(ct * ct, axis=-1, keepdims=True)
    sq_im = jnp.sum(ct_im * ct_im, axis=-1, keepdims=True)
    ones = jnp.ones((N, 1), jnp.float32)
    xa = jnp.concatenate([ct_im * (-2.0), sq_im, ones], axis=1)
    ya = jnp.concatenate([ct, ones, sq_b], axis=1)
    dists = _mmT(xa, ya, _HI)
    r256 = jax.lax.broadcasted_iota(jnp.int32, (N, N), 0)
    c256 = jax.lax.broadcasted_iota(jnp.int32, (N, N), 1)
    offblock = jnp.bitwise_and(r256, 7) != jax.lax.shift_right_logical(c256, 5)
    dists = jnp.where(offblock, _BIG, dists)

    c32 = jax.lax.broadcasted_iota(jnp.int32, (N, A), 1)
    grp_im = jnp.zeros((N, A), jnp.float32)
    for _ in range(M_GROUP):
        idx = jnp.argmin(dists, axis=-1).astype(jnp.int32)[:, None]
        dists = jnp.where(c256 == idx, _BIG, dists)
        j = jnp.bitwise_and(idx, 31)
        grp_im = grp_im + jnp.where(c32 == j, 1.0, 0.0)

    tri = jnp.where(
        jax.lax.broadcasted_iota(jnp.int32, (A, A), 0)
        <= jax.lax.broadcasted_iota(jnp.int32, (A, A), 1), 1.0, 0.0)
    pos = _mm(grp_im, tri)
    slot_ref[0:N, :] = pos * grp_im - (1.0 - grp_im)
    slot_ref[N:N + 8, :] = jnp.full((8, A), -1.0, jnp.float32)
    ii_im = _mm(pm, isin)
    ii_ref[...] = ii_im

    grp_out[...] = _mTm(pm, grp_im * ii_im)

    nt_ref[...] = ct

    r64 = jax.lax.broadcasted_iota(jnp.int32, (8 * M_GROUP, N), 0)
    c64 = jax.lax.broadcasted_iota(jnp.int32, (8 * M_GROUP, N), 1)
    tau_f = jax.lax.shift_right_logical(r64, 3).astype(jnp.float32)
    blockdiag = jnp.bitwise_and(r64, 7) == jax.lax.shift_right_logical(c64, 5)
    lane512 = jax.lax.broadcasted_iota(jnp.int32, (1, 4 * THOUGHT), 1)
    fwd512 = jnp.bitwise_and(jax.lax.shift_right_logical(lane512, 6), 1) == 0
    lane128 = jax.lax.broadcasted_iota(jnp.int32, (1, THOUGHT), 1)
    fwd128 = lane128 < H

    wih = wih_ref[...]
    whh = whh_ref[...]
    bc = bc_ref[...]

    def build_mt(sl):
        sl_exp = jnp.tile(jnp.tile(sl, (8, 1)), (1, 8))
        return jnp.where((sl_exp - 1.0 == tau_f) & blockdiag, 1.0, 0.0)

    th0 = _mm(build_mt(slot_ref[0:8, :]), ct, _HI)

    def step(i, th):
        sl = slot_ref[pl.ds(8 * i, 8), :]
        sln = slot_ref[pl.ds(8 * i + 8, 8), :]
        ii = ii_ref[pl.ds(8 * i, 8), :]
        nt = nt_ref[...]
        mt = build_mt(sl)
        mtn = build_mt(sln)

        xall = _mm(th, wih) + bc

        c = jnp.zeros((8, THOUGHT), jnp.float32)
        hh = jnp.zeros((8, THOUGHT), jnp.float32)
        hs = []
        pre_blks = []
        cc = None
        for t in range(M_GROUP):
            xf = xall[8 * t:8 * t + 8, :]
            xr = xall[8 * (7 - t):8 * (7 - t) + 8, :]
            g = jnp.where(fwd512, xf, xr)
            if t > 0:
                g = g + _mm(hh, whh)
            pre_blks.append(_mm(mtn[8 * t:8 * t + 8, :], nt, _HI))
            if t == 4:
                cc = _mmT(mtn, mt)
            ig = jax.nn.sigmoid(g[:, 0:128])
            fg = jax.nn.sigmoid(g[:, 128:256])
            gg = jnp.tanh(g[:, 256:384])
            og = jax.nn.sigmoid(g[:, 384:512])
            c = fg * c + ig * gg if t > 0 else ig * gg
            hh = og * jnp.tanh(c)
            hs.append(hh)
        integ = jnp.concatenate(
            [jnp.where(fwd128, hs[t], hs[7 - t]) for t in range(M_GROUP)],
            axis=0)

        delta = (integ - th) * jnp.tile(ii, (8, 1))
        nt_ref[...] = nt + _mTm(mt, delta, _HI)

        pre = jnp.concatenate(pre_blks, axis=0)
        return pre + _mm(cc, delta, _HI)

    jax.lax.fori_loop(0, A, step, th0)

    nt = nt_ref[...]
    nt_out[...] = nt

    xc = jax.nn.relu(jnp.concatenate([ct, nt], axis=1))
    y = _ln_in(_mm(xc, c1t_ref[...]) + c1b_ref[...], c1g_ref[...], c1bt_ref[...])
    z = _ln_in(_mm(y, c2t_ref[...]) + c2b_ref[...], c2g_ref[...], c2bt_ref[...])
    act_out[...] = jnp.tanh(z)


@jax.jit
def kernel(obs, a1_w1, a1_b1, a1_g1, a1_bt1, a1_w2, a1_b2, a1_g2, a1_bt2,
           a2_w1, a2_b1, a2_g1, a2_bt1, a2_w2, a2_b2, a2_g2, a2_bt2,
           att_w1, att_b1, att_w2, att_b2, att_w3, att_b3,
           wih_f, whh_f, bih_f, bhh_f, wih_r, whh_r, bih_r, bhh_r):
    obs_f = obs.reshape(N, OBS_DIM)

    def row(v):
        return v.reshape(1, -1)

    wft, wrt = wih_f.T, wih_r.T
    wih_c = jnp.concatenate(
        [jnp.concatenate([wft[:, 64 * k:64 * (k + 1)],
                          wrt[:, 64 * k:64 * (k + 1)]], axis=1)
         for k in range(4)], axis=1)
    hft, hrt = whh_f.T, whh_r.T
    zz = jnp.zeros((H, H), jnp.float32)
    whh_c = jnp.concatenate(
        [jnp.concatenate(
            [jnp.concatenate([hft[:, 64 * k:64 * (k + 1)], zz], axis=1),
             jnp.concatenate([zz, hrt[:, 64 * k:64 * (k + 1)]], axis=1)],
            axis=0)
         for k in range(4)], axis=1)
    bf = bih_f + bhh_f
    br = bih_r + bhh_r
    bc = jnp.concatenate(
        [jnp.concatenate([bf[64 * k:64 * (k + 1)], br[64 * k:64 * (k + 1)]])
         for k in range(4)]).reshape(1, -1)

    pm_np = np.zeros((N, N), np.float32)
    rp = np.arange(N)
    pm_np[rp, 32 * (rp % 8) + rp // 8] = 1.0
    pm = jnp.asarray(pm_np)

    out_shapes = [
        jax.ShapeDtypeStruct((N, ACT_DIM), jnp.float32),
        jax.ShapeDtypeStruct((N, A), jnp.float32),
        jax.ShapeDtypeStruct((N, 1), jnp.float32),
        jax.ShapeDtypeStruct((N, 1), jnp.float32),
        jax.ShapeDtypeStruct((N, THOUGHT), jnp.float32),
        jax.ShapeDtypeStruct((N, THOUGHT), jnp.float32),
    ]
    act, grp, ip, isin, nt, ct = pl.pallas_call(
        _fused_kernel,
        out_shape=out_shapes,
        scratch_shapes=[
            pltpu.VMEM((N, THOUGHT), jnp.float32),
            pltpu.VMEM((N + 8, A), jnp.float32),
            pltpu.VMEM((N, 1), jnp.float32),
        ],
    )(obs_f, a1_w1.T, row(a1_b1), row(a1_g1), row(a1_bt1),
      a1_w2.T, row(a1_b2), row(a1_g2), row(a1_bt2),
      att_w1.T, row(att_b1), att_w2.T, row(att_b2), att_w3.T, row(att_b3),
      wih_c, whh_c, bc,
      a2_w1.T, row(a2_b1), row(a2_g1), row(a2_bt1),
      a2_w2.T, row(a2_b2), row(a2_g2), row(a2_bt2),
      pm)

    return (act.reshape(B, A, ACT_DIM), grp.reshape(B, A, A),
            ip.reshape(B, A), (isin > 0.5).reshape(B, A),
            nt.reshape(B, A, THOUGHT), ct.reshape(B, A, THOUGHT))
